# baseline (speedup 1.0000x reference)
import functools

import jax
import jax.numpy as jnp
from jax import lax
from jax.experimental import pallas as pl
from jax.experimental.pallas import tpu as pltpu
from jax.experimental.pallas import tpu_sc as plsc

_B = 16384
_DIM = 64
_MARGIN = 1.0

_NC = 2
_NS = 16
_NW = _NC * _NS

_CHUNK = 128


def _sc_gather(ent_idx, rel_idx, ent_embs, rel_emb):
    n_ent_rows = ent_idx.shape[0]
    n_rel_rows = rel_idx.shape[0]
    ent_per_w = n_ent_rows // _NW
    rel_per_w = n_rel_rows // _NW
    mesh = plsc.VectorSubcoreMesh(core_axis_name="c", subcore_axis_name="s")

    @functools.partial(
        pl.kernel,
        out_type=(
            jax.ShapeDtypeStruct((n_ent_rows, _DIM), jnp.float32),
            jax.ShapeDtypeStruct((n_rel_rows, _DIM), jnp.float32),
        ),
        mesh=mesh,
        scratch_types=[
            pltpu.VMEM((_CHUNK,), jnp.int32),
            pltpu.VMEM((_CHUNK, _DIM), jnp.float32),
            pltpu.SemaphoreType.DMA,
        ],
        compiler_params=pltpu.CompilerParams(use_tc_tiling_on_sc=False),
    )
    def gather_kernel(ent_idx_hbm, rel_idx_hbm, ent_hbm, rel_hbm,
                      ent_out, rel_out, idx_v, rows_v, sem):
        wid = lax.axis_index("s") * _NC + lax.axis_index("c")

        ent_base = wid * ent_per_w
        for c in range(ent_per_w // _CHUNK):
            off = ent_base + c * _CHUNK
            pltpu.sync_copy(ent_idx_hbm.at[pl.ds(off, _CHUNK)], idx_v)
            pltpu.async_copy(ent_hbm.at[idx_v], rows_v, sem).wait()
            pltpu.sync_copy(rows_v, ent_out.at[pl.ds(off, _CHUNK)])

        rel_base = wid * rel_per_w
        for c in range(rel_per_w // _CHUNK):
            off = rel_base + c * _CHUNK
            pltpu.sync_copy(rel_idx_hbm.at[pl.ds(off, _CHUNK)], idx_v)
            pltpu.async_copy(rel_hbm.at[idx_v], rows_v, sem).wait()
            pltpu.sync_copy(rows_v, rel_out.at[pl.ds(off, _CHUNK)])

    return gather_kernel(ent_idx, rel_idx, ent_embs, rel_emb)


_BLK = 2048


def _dense_body(hp_ref, tp_ref, hn_ref, tn_ref, r_ref, out_ref):
    i = pl.program_id(0)

    def nrm(x):
        n = jnp.sqrt(jnp.sum(x * x, axis=1, keepdims=True))
        return x / jnp.maximum(n, 1e-12)

    hp = nrm(hp_ref[...])
    tp = nrm(tp_ref[...])
    hn = nrm(hn_ref[...])
    tn = nrm(tn_ref[...])
    r = nrm(r_ref[...])
    pos_e = jnp.sqrt(jnp.sum((hp + r - tp) ** 2, axis=1))
    neg_e = jnp.sqrt(jnp.sum((hn + r - tn) ** 2, axis=1))
    loss = jnp.maximum(_MARGIN + pos_e - neg_e, 0.0)
    s = jnp.sum(loss)

    @pl.when(i == 0)
    def _init():
        out_ref[0, 0] = s

    @pl.when(i != 0)
    def _acc():
        out_ref[0, 0] += s

    @pl.when(i == pl.num_programs(0) - 1)
    def _final():
        out_ref[0, 0] = out_ref[0, 0] / _B


def _dense_loss(hp, tp, hn, tn, r):
    grid = _B // _BLK
    row_spec = pl.BlockSpec((_BLK, _DIM), lambda i: (i, 0))
    return pl.pallas_call(
        _dense_body,
        grid=(grid,),
        in_specs=[row_spec] * 5,
        out_specs=pl.BlockSpec((1, 1), lambda i: (0, 0),
                               memory_space=pltpu.SMEM),
        out_shape=jax.ShapeDtypeStruct((1, 1), jnp.float32),
    )(hp, tp, hn, tn, r)


def kernel(pos_pairs, neg_pairs, rels, ent_embs, alignments, rel_emb):
    ent_idx = jnp.concatenate(
        [pos_pairs[:, 0], pos_pairs[:, 1], neg_pairs[:, 0], neg_pairs[:, 1]]
    )
    rel_idx = rels[:, 0]
    ent_rows, rel_rows = _sc_gather(ent_idx, rel_idx, ent_embs, rel_emb)
    hp = ent_rows[0:_B]
    tp = ent_rows[_B:2 * _B]
    hn = ent_rows[2 * _B:3 * _B]
    tn = ent_rows[3 * _B:4 * _B]
    out = _dense_loss(hp, tp, hn, tn, rel_rows)
    return out[0, 0]


# speedup vs baseline: 1.3599x; 1.3599x over previous
import functools

import jax
import jax.numpy as jnp
from jax import lax
from jax.experimental import pallas as pl
from jax.experimental.pallas import tpu as pltpu
from jax.experimental.pallas import tpu_sc as plsc

_B = 16384
_DIM = 64
_MARGIN = 1.0

_NC = 2
_NS = 16
_NW = _NC * _NS

_CHUNK = 512
_GRP = 16


def _sc_gather(ent_idx, rel_idx, ent_embs, rel_emb):
    n_ent_rows = ent_idx.shape[0]
    n_rel_rows = rel_idx.shape[0]
    ent_per_w = n_ent_rows // _NW
    rel_per_w = n_rel_rows // _NW
    mesh = plsc.VectorSubcoreMesh(core_axis_name="c", subcore_axis_name="s")

    @functools.partial(
        pl.kernel,
        out_type=(
            jax.ShapeDtypeStruct((n_ent_rows, _DIM), jnp.float32),
            jax.ShapeDtypeStruct((n_rel_rows, _DIM), jnp.float32),
        ),
        mesh=mesh,
        scratch_types=[
            pltpu.VMEM((_CHUNK,), jnp.int32),
            pltpu.VMEM((_CHUNK, _DIM), jnp.float32),
            pltpu.SemaphoreType.DMA,
        ],
    )
    def gather_kernel(ent_idx_hbm, rel_idx_hbm, ent_hbm, rel_hbm,
                      ent_out, rel_out, idx_v, rows_v, sem):
        wid = lax.axis_index("s") * _NC + lax.axis_index("c")

        def do_table(idx_hbm, tab_hbm, out_hbm, per_w):
            base = wid * per_w

            def chunk_body(c, carry):
                off = base + c * _CHUNK
                pltpu.sync_copy(idx_hbm.at[pl.ds(off, _CHUNK)], idx_v)

                def grp_body(g, carry):
                    gbase = pl.multiple_of(g * _GRP, _GRP)
                    vec = idx_v[pl.ds(gbase, _GRP)]
                    copies = []
                    for l in range(_GRP):
                        i = vec[l]
                        cp = pltpu.make_async_copy(
                            tab_hbm.at[pl.ds(i, 1), :],
                            rows_v.at[pl.ds(gbase + l, 1), :],
                            sem,
                        )
                        cp.start()
                        copies.append(cp)
                    for cp in copies:
                        cp.wait()
                    return carry

                lax.fori_loop(0, _CHUNK // _GRP, grp_body, 0)
                pltpu.sync_copy(rows_v, out_hbm.at[pl.ds(off, _CHUNK)])
                return carry

            lax.fori_loop(0, per_w // _CHUNK, chunk_body, 0)

        do_table(ent_idx_hbm, ent_hbm, ent_out, ent_per_w)
        do_table(rel_idx_hbm, rel_hbm, rel_out, rel_per_w)

    return gather_kernel(ent_idx, rel_idx, ent_embs, rel_emb)


_BLK = 2048


def _dense_body(hp_ref, tp_ref, hn_ref, tn_ref, r_ref, out_ref):
    i = pl.program_id(0)

    def nrm(x):
        n = jnp.sqrt(jnp.sum(x * x, axis=1, keepdims=True))
        return x / jnp.maximum(n, 1e-12)

    hp = nrm(hp_ref[...])
    tp = nrm(tp_ref[...])
    hn = nrm(hn_ref[...])
    tn = nrm(tn_ref[...])
    r = nrm(r_ref[...])
    pos_e = jnp.sqrt(jnp.sum((hp + r - tp) ** 2, axis=1))
    neg_e = jnp.sqrt(jnp.sum((hn + r - tn) ** 2, axis=1))
    loss = jnp.maximum(_MARGIN + pos_e - neg_e, 0.0)
    s = jnp.sum(loss)

    @pl.when(i == 0)
    def _init():
        out_ref[0, 0] = s

    @pl.when(i != 0)
    def _acc():
        out_ref[0, 0] += s

    @pl.when(i == pl.num_programs(0) - 1)
    def _final():
        out_ref[0, 0] = out_ref[0, 0] / _B


def _dense_loss(hp, tp, hn, tn, r):
    grid = _B // _BLK
    row_spec = pl.BlockSpec((_BLK, _DIM), lambda i: (i, 0))
    return pl.pallas_call(
        _dense_body,
        grid=(grid,),
        in_specs=[row_spec] * 5,
        out_specs=pl.BlockSpec((1, 1), lambda i: (0, 0),
                               memory_space=pltpu.SMEM),
        out_shape=jax.ShapeDtypeStruct((1, 1), jnp.float32),
    )(hp, tp, hn, tn, r)


def kernel(pos_pairs, neg_pairs, rels, ent_embs, alignments, rel_emb):
    ent_idx = jnp.concatenate(
        [pos_pairs[:, 0], pos_pairs[:, 1], neg_pairs[:, 0], neg_pairs[:, 1]]
    )
    rel_idx = rels[:, 0]
    ent_rows, rel_rows = _sc_gather(ent_idx, rel_idx, ent_embs, rel_emb)
    hp = ent_rows[0:_B]
    tp = ent_rows[_B:2 * _B]
    hn = ent_rows[2 * _B:3 * _B]
    tn = ent_rows[3 * _B:4 * _B]
    out = _dense_loss(hp, tp, hn, tn, rel_rows)
    return out[0, 0]
